# Initial kernel scaffold; baseline (speedup 1.0000x reference)
import jax
import jax.numpy as jnp
from jax.experimental import pallas as pl


def kernel(loc_pred, conf_pred, anchors, gt_boxes, gt_labels):
    raise NotImplementedError("write your pallas kernel here")


# speedup vs baseline: 35.9964x; 35.9964x over previous
import functools
import math

import jax
import jax.numpy as jnp
import numpy as np
from jax.experimental import pallas as pl
from jax.experimental.pallas import tpu as pltpu

IOU_THR = 0.5
NEG_RATIO = 3.0


def _mbl_kernel(conf_ref, loc_ref, anc_ref, gtb_ref, glab_ref, out_ref,
                t0_ref, t1_ref, t2_ref, t3_ref, tlab_ref,
                *, N, C, G, R):
    f32 = jnp.float32
    row_i = jax.lax.broadcasted_iota(jnp.int32, (R, 128), 0)
    lane_i = jax.lax.broadcasted_iota(jnp.int32, (R, 128), 1)
    flatidx = row_i * 128 + lane_i
    valid = flatidx < N

    acx = anc_ref[0]
    acy = anc_ref[1]
    aw = anc_ref[2]
    ah = anc_ref[3]
    ax1 = acx - aw * 0.5
    ay1 = acy - ah * 0.5
    ax2 = acx + aw * 0.5
    ay2 = acy + ah * 0.5
    area_a = (ax2 - ax1) * (ay2 - ay1)
    log_aw = jnp.log(aw)
    log_ah = jnp.log(ah)

    best_ov = jnp.full((R, 128), -1.0, f32)
    b_cx = jnp.zeros((R, 128), f32)
    b_cy = jnp.zeros((R, 128), f32)
    b_w = jnp.ones((R, 128), f32)
    b_h = jnp.ones((R, 128), f32)
    b_lab = jnp.zeros((R, 128), f32)
    bpi = []
    gbox = []
    for g in range(G):
        bx = gtb_ref[0, 0, 4 * g + 0]
        by = gtb_ref[0, 0, 4 * g + 1]
        bw = gtb_ref[0, 0, 4 * g + 2]
        bh = gtb_ref[0, 0, 4 * g + 3]
        labf = (glab_ref[0, 0, g] + 1).astype(f32)
        gx1 = bx - bw * 0.5
        gy1 = by - bh * 0.5
        gx2 = bx + bw * 0.5
        gy2 = by + bh * 0.5
        w = jnp.clip(jnp.minimum(gx2, ax2) - jnp.maximum(gx1, ax1), 0.0, None)
        h = jnp.clip(jnp.minimum(gy2, ay2) - jnp.maximum(gy1, ay1), 0.0, None)
        inter = w * h
        union = area_a + ((gx2 - gx1) * (gy2 - gy1)) - inter
        iou = inter / jnp.clip(union, 1e-10, None)

        upd = iou > best_ov
        best_ov = jnp.where(upd, iou, best_ov)
        b_cx = jnp.where(upd, bx, b_cx)
        b_cy = jnp.where(upd, by, b_cy)
        b_w = jnp.where(upd, bw, b_w)
        b_h = jnp.where(upd, bh, b_h)
        b_lab = jnp.where(upd, labf, b_lab)

        mx = jnp.max(iou)
        bpi_g = jnp.min(jnp.where(iou == mx, flatidx, N))
        bpi.append(bpi_g)
        gbox.append((bx, by, bw, bh, labf))

    over = best_ov > IOU_THR
    e0 = (b_cx - acx) / aw
    e1 = (b_cy - acy) / ah
    e2 = jnp.log(b_w) - log_aw
    e3 = jnp.log(b_h) - log_ah
    zero = jnp.zeros((R, 128), f32)
    t0_ref[...] = jnp.where(over, e0, zero)
    t1_ref[...] = jnp.where(over, e1, zero)
    t2_ref[...] = jnp.where(over, e2, zero)
    t3_ref[...] = jnp.where(over, e3, zero)
    tlab_ref[...] = jnp.where(over, b_lab, zero)

    lane1 = jax.lax.broadcasted_iota(jnp.int32, (1, 128), 1)
    for g in range(G):
        bx, by, bw, bh, labf = gbox[g]
        r_g = bpi[g] // 128
        l_g = bpi[g] % 128
        racx = anc_ref[0, pl.ds(r_g, 1), :]
        racy = anc_ref[1, pl.ds(r_g, 1), :]
        raw = anc_ref[2, pl.ds(r_g, 1), :]
        rah = anc_ref[3, pl.ds(r_g, 1), :]
        f0 = (bx - racx) / raw
        f1 = (by - racy) / rah
        f2 = jnp.log(jnp.full((1, 128), bw, f32)) - jnp.log(raw)
        f3 = jnp.log(jnp.full((1, 128), bh, f32)) - jnp.log(rah)
        lm = lane1 == l_g
        t0_ref[pl.ds(r_g, 1), :] = jnp.where(lm, f0, t0_ref[pl.ds(r_g, 1), :])
        t1_ref[pl.ds(r_g, 1), :] = jnp.where(lm, f1, t1_ref[pl.ds(r_g, 1), :])
        t2_ref[pl.ds(r_g, 1), :] = jnp.where(lm, f2, t2_ref[pl.ds(r_g, 1), :])
        t3_ref[pl.ds(r_g, 1), :] = jnp.where(lm, f3, t3_ref[pl.ds(r_g, 1), :])
        tlab_ref[pl.ds(r_g, 1), :] = jnp.where(
            lm, jnp.full((1, 128), labf, f32), tlab_ref[pl.ds(r_g, 1), :])

    tlab = tlab_ref[...]
    pos = tlab > 0.0
    posf = pos.astype(f32)
    npos = jnp.sum(posf)

    lsum = f32(0)
    for c, t_ref in enumerate((t0_ref, t1_ref, t2_ref, t3_ref)):
        d = jnp.abs(loc_ref[0, c] - t_ref[...])
        sl = jnp.where(d < 1.0, 0.5 * d * d, d - 0.5)
        lsum = lsum + jnp.sum(jnp.where(pos, sl, zero))

    m = conf_ref[0, 0]
    for c in range(1, C):
        m = jnp.maximum(m, conf_ref[0, c])
    s = jnp.exp(conf_ref[0, 0] - m)
    confL = jnp.zeros((R, 128), f32)
    for c in range(1, C):
        cc = conf_ref[0, c]
        s = s + jnp.exp(cc - m)
        confL = jnp.where(tlab == f32(c), cc, confL)
    lse = m + jnp.log(s)
    conf0 = conf_ref[0, 0]

    posce = jnp.sum(jnp.where(pos, lse - confL, zero))

    q = jnp.where(pos | jnp.logical_not(valid), zero, lse - conf0)
    qi = jax.lax.bitcast_convert_type(q, jnp.int32)
    kneg_f = jnp.minimum(NEG_RATIO * npos, f32(N - 1))
    k = kneg_f.astype(jnp.int32)

    t = jnp.int32(0)
    for bit in range(30, -1, -1):
        trial = t | jnp.int32(1 << bit)
        cnt = jnp.sum((qi >= trial).astype(jnp.int32))
        t = jnp.where(cnt >= k, trial, t)
    tau_i = t
    tau_f = jax.lax.bitcast_convert_type(tau_i, f32)
    gt_mask = qi > tau_i
    cnt_gt = jnp.sum(gt_mask.astype(jnp.int32))
    sum_gt = jnp.sum(jnp.where(gt_mask, q, zero))
    need_eq = k - cnt_gt
    topk = sum_gt + need_eq.astype(f32) * tau_f

    zeros_m = (qi == 0) & valid
    mzi = jnp.int32(0)
    for bit in range(14, -1, -1):
        trial = mzi | jnp.int32(1 << bit)
        cntz = jnp.sum((zeros_m & (flatidx < trial)).astype(jnp.int32))
        mzi = jnp.where(cntz < need_eq, trial, mzi)
    pos_in = jnp.sum((pos & (flatidx <= mzi)).astype(jnp.int32))
    pos_in = jnp.where((tau_i == 0) & (need_eq > 0), pos_in, 0)

    nsamp = npos + kneg_f - pos_in.astype(f32)

    lane = jax.lax.broadcasted_iota(jnp.int32, (1, 128), 1)
    vec = jnp.where(lane == 0, lsum, 0.0)
    vec = jnp.where(lane == 1, posce, vec)
    vec = jnp.where(lane == 2, topk, vec)
    vec = jnp.where(lane == 3, npos, vec)
    vec = jnp.where(lane == 4, nsamp, vec)
    out_ref[0] = vec


def kernel(loc_pred, conf_pred, anchors, gt_boxes, gt_labels):
    B, N, C = conf_pred.shape
    G = gt_boxes.shape[1]
    NP = ((N + 127) // 128) * 128
    R = NP // 128
    padn = NP - N

    conf_t = jnp.pad(conf_pred, ((0, 0), (0, padn), (0, 0)))
    conf_t = conf_t.transpose(0, 2, 1).reshape(B, C, R, 128)
    loc_t = jnp.pad(loc_pred, ((0, 0), (0, padn), (0, 0)))
    loc_t = loc_t.transpose(0, 2, 1).reshape(B, 4, R, 128)
    pad_rows = jnp.broadcast_to(
        jnp.array([-1000.0, -1000.0, 1.0, 1.0], jnp.float32), (padn, 4))
    anc_t = jnp.concatenate([anchors, pad_rows], 0).T.reshape(4, R, 128)
    gtb = gt_boxes.reshape(B, 1, 4 * G)
    glab = gt_labels.reshape(B, 1, G)

    partial = pl.pallas_call(
        functools.partial(_mbl_kernel, N=N, C=C, G=G, R=R),
        grid=(B,),
        in_specs=[
            pl.BlockSpec((1, C, R, 128), lambda b: (b, 0, 0, 0)),
            pl.BlockSpec((1, 4, R, 128), lambda b: (b, 0, 0, 0)),
            pl.BlockSpec((4, R, 128), lambda b: (0, 0, 0)),
            pl.BlockSpec((1, 1, 4 * G), lambda b: (b, 0, 0),
                         memory_space=pltpu.SMEM),
            pl.BlockSpec((1, 1, G), lambda b: (b, 0, 0),
                         memory_space=pltpu.SMEM),
        ],
        out_specs=pl.BlockSpec((1, 1, 128), lambda b: (b, 0, 0)),
        out_shape=jax.ShapeDtypeStruct((B, 1, 128), jnp.float32),
        scratch_shapes=[pltpu.VMEM((R, 128), jnp.float32)] * 5,
        compiler_params=pltpu.CompilerParams(
            dimension_semantics=("arbitrary",)),
    )(conf_t, loc_t, anc_t, gtb, glab)

    loc_sum = jnp.sum(partial[:, 0, 0])
    posce = jnp.sum(partial[:, 0, 1])
    topk = jnp.sum(partial[:, 0, 2])
    npos = jnp.sum(partial[:, 0, 3])
    nsamp = jnp.sum(partial[:, 0, 4])
    log_c = np.float32(math.log(float(C)))
    ce = posce + topk + (f32_const(B * N) - nsamp) * log_c
    return (loc_sum + ce) / npos


def f32_const(x):
    return jnp.float32(x)
